# baseline (speedup 1.0000x reference)
import jax
import jax.numpy as jnp
from jax import lax
from jax.experimental import pallas as pl
from jax.experimental.pallas import tpu as pltpu, tpu_sc as plsc

_D = 128
_SCALE = float(_D) ** 0.5
_NC = 2
_NS = 16
_NW = _NC * _NS

_B = 4096 * 200
_BPW = _B // _NW
_CHUNK = 80
_NCHUNK = _BPW // _CHUNK
_NBUF = 8
_PF = 6


def _gather_body(x_hbm, wte_hbm, out_hbm, idx_v, rows, gsems, ssems):
    wid = lax.axis_index("s") * _NC + lax.axis_index("c")
    base = wid * _BPW
    pltpu.sync_copy(x_hbm.at[pl.ds(base, _BPW)], idx_v)

    def start_gather(c, b):
        pltpu.async_copy(wte_hbm.at[idx_v.at[pl.ds(c * _CHUNK, _CHUNK)]],
                         rows[b], gsems[b])

    def wait_gather(b):
        pltpu.make_async_copy(wte_hbm.at[idx_v.at[pl.ds(0, _CHUNK)]],
                              rows[b], gsems[b]).wait()

    def start_store(c, b):
        pltpu.async_copy(rows[b], out_hbm.at[pl.ds(base + c * _CHUNK, _CHUNK)],
                         ssems[b])

    def wait_store(c, b):
        pltpu.make_async_copy(rows[b],
                              out_hbm.at[pl.ds(base + c * _CHUNK, _CHUNK)],
                              ssems[b]).wait()

    def scale(b):
        def srow(r, carry):
            for u in range(2):
                for j in range(_D // 16):
                    sl = (r * 2 + u, pl.ds(j * 16, 16))
                    rows[b][sl] = rows[b][sl] * _SCALE
            return carry
        lax.fori_loop(0, _CHUNK // 2, srow, 0)

    for c in range(_PF):
        start_gather(c, c)

    def step(s, carry):
        for i in range(_NBUF):
            c = s * _NBUF + i
            b = i
            bpf = (i + _PF) % _NBUF

            @pl.when(jnp.logical_and(c + _PF < _NCHUNK, c >= _NBUF - _PF))
            def _():
                wait_store(c + _PF - _NBUF, bpf)

            @pl.when(c + _PF < _NCHUNK)
            def _():
                start_gather(c + _PF, bpf)

            wait_gather(b)
            scale(b)
            start_store(c, b)
        return carry

    lax.fori_loop(0, _NCHUNK // _NBUF, step, 0)

    for i in range(_NBUF):
        c = _NCHUNK - _NBUF + i
        wait_store(c, c % _NBUF)


_sc_gather = pl.kernel(
    _gather_body,
    out_type=jax.ShapeDtypeStruct((_B, _D), jnp.float32),
    mesh=plsc.VectorSubcoreMesh(core_axis_name="c", subcore_axis_name="s"),
    scratch_types=[
        pltpu.VMEM((_BPW,), jnp.int32),
        [pltpu.VMEM((_CHUNK, _D), jnp.float32) for _ in range(_NBUF)],
        [pltpu.SemaphoreType.DMA for _ in range(_NBUF)],
        [pltpu.SemaphoreType.DMA for _ in range(_NBUF)],
    ],
)


def kernel(X, wte):
    n, t = X.shape
    x_flat = X.reshape(n * t).astype(jnp.int32)
    out = _sc_gather(x_flat, wte)
    return out.reshape(n, t, _D)
